# baseline (speedup 1.0000x reference)
import jax
import jax.numpy as jnp
from jax import lax
from jax.experimental import pallas as pl
from jax.experimental.pallas import tpu as pltpu
from jax.experimental.pallas import tpu_sc as plsc

_NUM_CORES = 2
_NUM_SUBCORES = 16
_NUM_WORKERS = _NUM_CORES * _NUM_SUBCORES
_CHUNK = 32


def _make_body(batch, seq, width, rows_per_worker, n_chunks):
    def body(pe_hbm, out_hbm, buf0, buf1, rsem0, rsem1, wsem0, wsem1):
        wid = lax.axis_index("s") * _NUM_CORES + lax.axis_index("c")
        base = wid * rows_per_worker
        bufs, rsems, wsems = (buf0, buf1), (rsem0, rsem1), (wsem0, wsem1)

        def start_read(j):
            return pltpu.async_copy(
                pe_hbm.at[pl.ds(base + j * _CHUNK, _CHUNK)], bufs[j % 2],
                rsems[j % 2])

        r_handles = [None, None]
        w_handles = [[], []]
        r_handles[0] = start_read(0)
        for j in range(n_chunks):
            p = j % 2
            r_handles[p].wait()
            w = [pltpu.async_copy(
                     bufs[p],
                     out_hbm.at[pl.ds(b * seq + base + j * _CHUNK, _CHUNK)],
                     wsems[p])
                 for b in range(batch)]
            if j + 1 < n_chunks:
                for h in w_handles[(j + 1) % 2]:
                    h.wait()
                r_handles[(j + 1) % 2] = start_read(j + 1)
            w_handles[p] = w
        for p in range(2):
            for h in w_handles[p]:
                h.wait()
    return body


def kernel(inputs, position_embeddings):
    batch, seq, width = inputs.shape
    pe = position_embeddings[:seq, :]
    rows_per_worker = seq // _NUM_WORKERS
    n_chunks = rows_per_worker // _CHUNK
    run = pl.kernel(
        _make_body(batch, seq, width, rows_per_worker, n_chunks),
        out_type=jax.ShapeDtypeStruct((batch * seq, width), jnp.float32),
        mesh=plsc.VectorSubcoreMesh(core_axis_name="c", subcore_axis_name="s"),
        scratch_types=[
            pltpu.VMEM((_CHUNK, width), jnp.float32),
            pltpu.VMEM((_CHUNK, width), jnp.float32),
            pltpu.SemaphoreType.DMA,
            pltpu.SemaphoreType.DMA,
            pltpu.SemaphoreType.DMA,
            pltpu.SemaphoreType.DMA,
        ],
    )
    out = run(pe)
    return out.reshape(batch, seq, width)
